# baseline (speedup 1.0000x reference)
import functools

import jax
import jax.numpy as jnp
from jax import lax
from jax.experimental import pallas as pl
from jax.experimental.pallas import tpu as pltpu
from jax.experimental.pallas import tpu_sc as plsc

BATCH = 16384
EMB = 32
NC = 2
NS = 16
NW = NC * NS
BPW = BATCH // NW
UNROLL = 16
CHUNK = 256


def _sc_gather_body(user_h, item_h, ug_h, ig_h, um_h, im_h,
                    oug_h, oig_h, oum_h, oim_h,
                    uvmem, ivmem, buf0, buf1, sem, sem2, osem, osem2):
  wid = lax.axis_index("s") * NC + lax.axis_index("c")
  base = wid * BPW
  pltpu.sync_copy(user_h.at[pl.ds(base, BPW)], uvmem)
  pltpu.sync_copy(item_h.at[pl.ds(base, BPW)], ivmem)

  steps = []
  for tbl, idxs, out_h in ((ug_h, uvmem, oug_h), (ig_h, ivmem, oig_h),
                           (um_h, uvmem, oum_h), (im_h, ivmem, oim_h)):
    for h in range(BPW // CHUNK):
      steps.append((tbl, idxs, out_h, h))

  bufs = (buf0, buf1)
  sems = (sem, sem2)
  osems = (osem, osem2)
  for s, (tbl, idxs, out_h, h) in enumerate(steps):
    p = s % 2
    buf = bufs[p]
    if s >= 2:
      prev_out, prev_h = steps[s - 2][2], steps[s - 2][3]
      pltpu.make_async_copy(
          buf, prev_out.at[pl.ds(base + prev_h * CHUNK, CHUNK)],
          osems[p]).wait()

    def body(c, tbl=tbl, idxs=idxs, buf=buf, h=h, p=p):
      iv = idxs[pl.ds(h * CHUNK + c * UNROLL, UNROLL)]
      for r in range(UNROLL):
        pltpu.async_copy(tbl.at[pl.ds(iv[r], 1)],
                         buf.at[pl.ds(c * UNROLL + r, 1)], sems[p])
      return 0
    lax.fori_loop(0, CHUNK // UNROLL, lambda c, _, b=body: b(c), 0,
                  unroll=False)
    pltpu.make_async_copy(tbl.at[pl.ds(0, CHUNK)], buf, sems[p]).wait()
    pltpu.async_copy(buf, out_h.at[pl.ds(base + h * CHUNK, CHUNK)],
                     osems[p])
  for s in (len(steps) - 2, len(steps) - 1):
    tbl, idxs, out_h, h = steps[s]
    pltpu.make_async_copy(bufs[s % 2],
                          out_h.at[pl.ds(base + h * CHUNK, CHUNK)],
                          osems[s % 2]).wait()


@functools.lru_cache(maxsize=None)
def _sc_gather():
  return functools.partial(
      pl.kernel,
      out_type=(jax.ShapeDtypeStruct((BATCH, EMB), jnp.float32),) * 4,
      mesh=plsc.VectorSubcoreMesh(core_axis_name="c", subcore_axis_name="s",
                                  num_cores=NC, num_subcores=NS),
      compiler_params=pltpu.CompilerParams(use_tc_tiling_on_sc=True),
      scratch_types=[
          pltpu.VMEM((BPW,), jnp.int32),
          pltpu.VMEM((BPW,), jnp.int32),
          pltpu.VMEM((CHUNK, EMB), jnp.float32),
          pltpu.VMEM((CHUNK, EMB), jnp.float32),
          pltpu.SemaphoreType.DMA,
          pltpu.SemaphoreType.DMA,
          pltpu.SemaphoreType.DMA,
          pltpu.SemaphoreType.DMA,
      ],
  )(_sc_gather_body)


BLK = 2048


def _mlp_body(ug_ref, ig_ref, mu_ref, mi_ref,
              w1a_ref, w1b_ref, w2_ref, w3_ref, woa_ref, wob_ref,
              b1_ref, b2_ref, b3_ref, bo_ref, out_ref):
  mu = mu_ref[...]
  mi = mi_ref[...]
  h1 = jnp.dot(mu, w1a_ref[...], preferred_element_type=jnp.float32)
  h1 = h1 + jnp.dot(mi, w1b_ref[...], preferred_element_type=jnp.float32)
  h1 = jnp.maximum(h1 + b1_ref[...], 0.0)
  h2 = jnp.maximum(
      jnp.dot(h1, w2_ref[...], preferred_element_type=jnp.float32)
      + b2_ref[...], 0.0)
  h3 = jnp.maximum(
      jnp.dot(h2, w3_ref[...], preferred_element_type=jnp.float32)
      + b3_ref[...], 0.0)
  gmf = ug_ref[...] * ig_ref[...]
  logit = (jnp.dot(gmf, woa_ref[...], preferred_element_type=jnp.float32)
           + jnp.dot(h3, wob_ref[...], preferred_element_type=jnp.float32)
           + bo_ref[...])
  out_ref[...] = 1.0 / (1.0 + jnp.exp(-logit))


def _mlp_call(ug, ig, mu, mi, w1a, w1b, w2t, w3t, woa, wob, b1, b2, b3, bo):
  grid = (BATCH // BLK,)
  bspec = pl.BlockSpec((BLK, EMB), lambda i: (i, 0))
  wspec = lambda shape: pl.BlockSpec(shape, lambda i: (0, 0))
  return pl.pallas_call(
      _mlp_body,
      grid=grid,
      in_specs=[bspec, bspec, bspec, bspec,
                wspec((EMB, 64)), wspec((EMB, 64)), wspec((64, 32)),
                wspec((32, 16)), wspec((EMB, 1)), wspec((16, 1)),
                wspec((1, 64)), wspec((1, 32)), wspec((1, 16)),
                wspec((1, 1))],
      out_specs=pl.BlockSpec((BLK, 1), lambda i: (i, 0)),
      out_shape=jax.ShapeDtypeStruct((BATCH, 1), jnp.float32),
  )(ug, ig, mu, mi, w1a, w1b, w2t, w3t, woa, wob, b1, b2, b3, bo)


def kernel(user, item, Ug, Ig, Um, Im, W1, b1, W2, b2, W3, b3, Wo, bo):
  user = user.astype(jnp.int32)
  item = item.astype(jnp.int32)
  ug, ig, mu, mi = _sc_gather()(user, item, Ug, Ig, Um, Im)
  w1a = W1[:, :EMB].T
  w1b = W1[:, EMB:].T
  w2t = W2.T
  w3t = W3.T
  woa = Wo[:, :EMB].T
  wob = Wo[:, EMB:].T
  out = _mlp_call(ug, ig, mu, mi, w1a, w1b, w2t, w3t, woa, wob,
                  b1.reshape(1, 64), b2.reshape(1, 32), b3.reshape(1, 16),
                  bo.reshape(1, 1))
  return jnp.squeeze(out)


# speedup vs baseline: 2.3773x; 2.3773x over previous
import functools

import jax
import jax.numpy as jnp
from jax import lax
from jax.experimental import pallas as pl
from jax.experimental.pallas import tpu as pltpu
from jax.experimental.pallas import tpu_sc as plsc

BATCH = 16384
EMB = 32
NC = 2
NS = 16
NW = NC * NS
BPW = BATCH // NW
GRP = 8
CHUNK = 128
LANES = 16


def _extract(slabs, g, iv, ext, k, sem):
  for r in range(GRP):
    col = jnp.broadcast_to(iv[r] & 127, (LANES,))
    rows0 = lax.iota(jnp.int32, LANES)
    sl = slabs.at[g * GRP + r]
    lo = plsc.load_gather(sl, [rows0, col])
    hi = plsc.load_gather(sl, [rows0 + LANES, col])
    ext[k + r, pl.ds(0, LANES)] = lo
    ext[k + r, pl.ds(LANES, LANES)] = hi


def _sc_gather_body(user_h, item_h, ug_h, ig_h, um_h, im_h,
                    oug_h, oig_h, oum_h, oim_h,
                    uvmem, ivmem, slabs, ext0, ext1, sem, osem0, osem1):
  wid = lax.axis_index("s") * NC + lax.axis_index("c")
  base = wid * BPW
  pltpu.sync_copy(user_h.at[pl.ds(base, BPW)], uvmem)
  pltpu.sync_copy(item_h.at[pl.ds(base, BPW)], ivmem)

  exts = (ext0, ext1)
  osems = (osem0, osem1)
  steps = []
  for tbl, idxs, out_h in ((ug_h, uvmem, oug_h), (ig_h, ivmem, oig_h),
                           (um_h, uvmem, oum_h), (im_h, ivmem, oim_h)):
    for h in range(BPW // CHUNK):
      steps.append((tbl, idxs, out_h, h))

  for s, (tbl, idxs, out_h, h) in enumerate(steps):
    p = s % 2
    ext = exts[p]
    if s >= 2:
      prev_out, prev_h = steps[s - 2][2], steps[s - 2][3]
      pltpu.make_async_copy(
          ext, prev_out.at[pl.ds(base + prev_h * CHUNK, CHUNK)],
          osems[p]).wait()

    def body(c, tbl=tbl, idxs=idxs, ext=ext):
      def fire(g, iv):
        for r in range(GRP):
          src_col = pl.multiple_of(iv[r] & ~jnp.int32(127), 128)
          pltpu.async_copy(tbl.at[:, pl.ds(src_col, 128)],
                           slabs.at[g * GRP + r], sem)

      k0 = c * (2 * GRP)
      iv16 = idxs[pl.ds(h * CHUNK + k0, 2 * GRP)]
      iv0 = [iv16[r] for r in range(GRP)]
      iv1 = [iv16[r + GRP] for r in range(GRP)]
      fire(0, iv0)
      fire(1, iv1)
      pltpu.make_async_copy(tbl.at[:, pl.ds(0, GRP * 128)],
                            slabs.at[pl.ds(0, GRP)], sem).wait()
      _extract(slabs, 0, iv0, ext, k0, sem)
      pltpu.make_async_copy(tbl.at[:, pl.ds(0, GRP * 128)],
                            slabs.at[pl.ds(GRP, GRP)], sem).wait()
      _extract(slabs, 1, iv1, ext, k0 + GRP, sem)
      return 0

    lax.fori_loop(0, CHUNK // (2 * GRP), lambda c, _, b=body: b(c), 0,
                  unroll=False)
    pltpu.async_copy(ext, out_h.at[pl.ds(base + h * CHUNK, CHUNK)],
                     osems[p])
  for s in (len(steps) - 2, len(steps) - 1):
    tbl, idxs, out_h, h = steps[s]
    pltpu.make_async_copy(exts[s % 2],
                          out_h.at[pl.ds(base + h * CHUNK, CHUNK)],
                          osems[s % 2]).wait()


@functools.lru_cache(maxsize=None)
def _sc_gather():
  return functools.partial(
      pl.kernel,
      out_type=(jax.ShapeDtypeStruct((BATCH, EMB), jnp.float32),) * 4,
      mesh=plsc.VectorSubcoreMesh(core_axis_name="c", subcore_axis_name="s",
                                  num_cores=NC, num_subcores=NS),
      compiler_params=pltpu.CompilerParams(use_tc_tiling_on_sc=True,
                                           needs_layout_passes=False),
      scratch_types=[
          pltpu.VMEM((BPW,), jnp.int32),
          pltpu.VMEM((BPW,), jnp.int32),
          pltpu.VMEM((2 * GRP, EMB, 128), jnp.float32),
          pltpu.VMEM((CHUNK, EMB), jnp.float32),
          pltpu.VMEM((CHUNK, EMB), jnp.float32),
          pltpu.SemaphoreType.DMA,
          pltpu.SemaphoreType.DMA,
          pltpu.SemaphoreType.DMA,
      ],
  )(_sc_gather_body)


BLK = 2048


def _mlp_body(ug_ref, ig_ref, mu_ref, mi_ref,
              w1a_ref, w1b_ref, w2_ref, w3_ref, woa_ref, wob_ref,
              b1_ref, b2_ref, b3_ref, bo_ref, out_ref):
  mu = mu_ref[...]
  mi = mi_ref[...]
  h1 = jnp.dot(mu, w1a_ref[...], preferred_element_type=jnp.float32)
  h1 = h1 + jnp.dot(mi, w1b_ref[...], preferred_element_type=jnp.float32)
  h1 = jnp.maximum(h1 + b1_ref[...], 0.0)
  h2 = jnp.maximum(
      jnp.dot(h1, w2_ref[...], preferred_element_type=jnp.float32)
      + b2_ref[...], 0.0)
  h3 = jnp.maximum(
      jnp.dot(h2, w3_ref[...], preferred_element_type=jnp.float32)
      + b3_ref[...], 0.0)
  gmf = ug_ref[...] * ig_ref[...]
  logit = (jnp.dot(gmf, woa_ref[...], preferred_element_type=jnp.float32)
           + jnp.dot(h3, wob_ref[...], preferred_element_type=jnp.float32)
           + bo_ref[...])
  out_ref[...] = 1.0 / (1.0 + jnp.exp(-logit))


def _mlp_call(ug, ig, mu, mi, w1a, w1b, w2t, w3t, woa, wob, b1, b2, b3, bo):
  grid = (BATCH // BLK,)
  bspec = pl.BlockSpec((BLK, EMB), lambda i: (i, 0))
  wspec = lambda shape: pl.BlockSpec(shape, lambda i: (0, 0))
  return pl.pallas_call(
      _mlp_body,
      grid=grid,
      in_specs=[bspec, bspec, bspec, bspec,
                wspec((EMB, 64)), wspec((EMB, 64)), wspec((64, 32)),
                wspec((32, 16)), wspec((EMB, 1)), wspec((16, 1)),
                wspec((1, 64)), wspec((1, 32)), wspec((1, 16)),
                wspec((1, 1))],
      out_specs=pl.BlockSpec((BLK, 1), lambda i: (i, 0)),
      out_shape=jax.ShapeDtypeStruct((BATCH, 1), jnp.float32),
  )(ug, ig, mu, mi, w1a, w1b, w2t, w3t, woa, wob, b1, b2, b3, bo)


def kernel(user, item, Ug, Ig, Um, Im, W1, b1, W2, b2, W3, b3, Wo, bo):
  user = user.astype(jnp.int32)
  item = item.astype(jnp.int32)
  ug, ig, mu, mi = _sc_gather()(user, item, Ug.T, Ig.T, Um.T, Im.T)
  w1a = W1[:, :EMB].T
  w1b = W1[:, EMB:].T
  w2t = W2.T
  w3t = W3.T
  woa = Wo[:, :EMB].T
  wob = Wo[:, EMB:].T
  out = _mlp_call(ug, ig, mu, mi, w1a, w1b, w2t, w3t, woa, wob,
                  b1.reshape(1, 64), b2.reshape(1, 32), b3.reshape(1, 16),
                  bo.reshape(1, 1))
  return jnp.squeeze(out)


# speedup vs baseline: 2.9439x; 1.2383x over previous
import functools

import jax
import jax.numpy as jnp
from jax import lax
from jax.experimental import pallas as pl
from jax.experimental.pallas import tpu as pltpu
from jax.experimental.pallas import tpu_sc as plsc

BATCH = 16384
EMB = 32
NC = 2
NS = 16
NW = NC * NS
BPW = BATCH // NW
GRP = 8
CHUNK = 128
LANES = 16


def _extract(slabs, g, iv, ext, k, sem):
  for r in range(GRP):
    col = jnp.broadcast_to(iv[r] & 127, (LANES,))
    rows0 = lax.iota(jnp.int32, LANES)
    sl = slabs.at[g * GRP + r]
    lo = plsc.load_gather(sl, [rows0, col])
    hi = plsc.load_gather(sl, [rows0 + LANES, col])
    ext[k + r, pl.ds(0, LANES)] = lo
    ext[k + r, pl.ds(LANES, LANES)] = hi


def _sc_gather_body(user_h, item_h, ug_h, ig_h, um_h, im_h,
                    oug_h, oig_h, oum_h, oim_h,
                    uvmem, ivmem, slabs, ext0, ext1, sem, osem0, osem1):
  wid = lax.axis_index("s") * NC + lax.axis_index("c")
  base = wid * BPW
  pltpu.sync_copy(user_h.at[pl.ds(base, BPW)], uvmem)
  pltpu.sync_copy(item_h.at[pl.ds(base, BPW)], ivmem)

  del ext1, osem0, osem1
  ext = ext0
  n_grp = BPW // (2 * GRP)
  grp_per_chunk = CHUNK // (2 * GRP)

  for tbl, idxs, out_h in ((ug_h, uvmem, oug_h), (ig_h, ivmem, oig_h),
                           (um_h, uvmem, oum_h), (im_h, ivmem, oim_h)):

    def fire(g, iv, tbl=tbl):
      for r in range(GRP):
        src_col = pl.multiple_of(iv[r] & ~jnp.int32(127), 128)
        pltpu.async_copy(tbl.at[:, pl.ds(src_col, 128)],
                         slabs.at[g * GRP + r], sem)

    def drain(g, tbl=tbl):
      pltpu.make_async_copy(tbl.at[:, pl.ds(0, GRP * 128)],
                            slabs.at[pl.ds(g * GRP, GRP)], sem).wait()

    iv_p = idxs[pl.ds(0, 2 * GRP)]
    fire(0, [iv_p[r] for r in range(GRP)])
    fire(1, [iv_p[r + GRP] for r in range(GRP)])

    def body(c, iv_pend, idxs=idxs, tbl=tbl, out_h=out_h):
      off = jnp.minimum(c * (2 * GRP), BPW - 2 * GRP)
      iv_new = idxs[pl.ds(off, 2 * GRP)]
      k0 = (c - 1) * (2 * GRP)
      krow = lax.rem(k0, CHUNK)
      drain(0)
      _extract(slabs, 0, [iv_pend[r] for r in range(GRP)], ext, krow, sem)

      @pl.when(c < n_grp)
      def _():
        fire(0, [iv_new[r] for r in range(GRP)])

      drain(1)
      _extract(slabs, 1, [iv_pend[r + GRP] for r in range(GRP)], ext,
               krow + GRP, sem)

      @pl.when(c < n_grp)
      def _():
        fire(1, [iv_new[r + GRP] for r in range(GRP)])

      @pl.when(lax.rem(c, grp_per_chunk) == 0)
      def _():
        chunk = c // grp_per_chunk - 1
        pltpu.sync_copy(ext,
                        out_h.at[pl.ds(base + chunk * CHUNK, CHUNK)])

      return iv_new

    lax.fori_loop(1, n_grp + 1, body, iv_p, unroll=False)


@functools.lru_cache(maxsize=None)
def _sc_gather():
  return functools.partial(
      pl.kernel,
      out_type=(jax.ShapeDtypeStruct((BATCH, EMB), jnp.float32),) * 4,
      mesh=plsc.VectorSubcoreMesh(core_axis_name="c", subcore_axis_name="s",
                                  num_cores=NC, num_subcores=NS),
      compiler_params=pltpu.CompilerParams(use_tc_tiling_on_sc=True,
                                           needs_layout_passes=False),
      scratch_types=[
          pltpu.VMEM((BPW,), jnp.int32),
          pltpu.VMEM((BPW,), jnp.int32),
          pltpu.VMEM((2 * GRP, EMB, 128), jnp.float32),
          pltpu.VMEM((CHUNK, EMB), jnp.float32),
          pltpu.VMEM((CHUNK, EMB), jnp.float32),
          pltpu.SemaphoreType.DMA,
          pltpu.SemaphoreType.DMA,
          pltpu.SemaphoreType.DMA,
      ],
  )(_sc_gather_body)


BLK = 2048


def _mlp_body(ug_ref, ig_ref, mu_ref, mi_ref,
              w1a_ref, w1b_ref, w2_ref, w3_ref, woa_ref, wob_ref,
              b1_ref, b2_ref, b3_ref, bo_ref, out_ref):
  mu = mu_ref[...]
  mi = mi_ref[...]
  h1 = jnp.dot(mu, w1a_ref[...], preferred_element_type=jnp.float32)
  h1 = h1 + jnp.dot(mi, w1b_ref[...], preferred_element_type=jnp.float32)
  h1 = jnp.maximum(h1 + b1_ref[...], 0.0)
  h2 = jnp.maximum(
      jnp.dot(h1, w2_ref[...], preferred_element_type=jnp.float32)
      + b2_ref[...], 0.0)
  h3 = jnp.maximum(
      jnp.dot(h2, w3_ref[...], preferred_element_type=jnp.float32)
      + b3_ref[...], 0.0)
  gmf = ug_ref[...] * ig_ref[...]
  logit = (jnp.dot(gmf, woa_ref[...], preferred_element_type=jnp.float32)
           + jnp.dot(h3, wob_ref[...], preferred_element_type=jnp.float32)
           + bo_ref[...])
  out_ref[...] = 1.0 / (1.0 + jnp.exp(-logit))


def _mlp_call(ug, ig, mu, mi, w1a, w1b, w2t, w3t, woa, wob, b1, b2, b3, bo):
  grid = (BATCH // BLK,)
  bspec = pl.BlockSpec((BLK, EMB), lambda i: (i, 0))
  wspec = lambda shape: pl.BlockSpec(shape, lambda i: (0, 0))
  return pl.pallas_call(
      _mlp_body,
      grid=grid,
      in_specs=[bspec, bspec, bspec, bspec,
                wspec((EMB, 64)), wspec((EMB, 64)), wspec((64, 32)),
                wspec((32, 16)), wspec((EMB, 1)), wspec((16, 1)),
                wspec((1, 64)), wspec((1, 32)), wspec((1, 16)),
                wspec((1, 1))],
      out_specs=pl.BlockSpec((BLK, 1), lambda i: (i, 0)),
      out_shape=jax.ShapeDtypeStruct((BATCH, 1), jnp.float32),
  )(ug, ig, mu, mi, w1a, w1b, w2t, w3t, woa, wob, b1, b2, b3, bo)


def kernel(user, item, Ug, Ig, Um, Im, W1, b1, W2, b2, W3, b3, Wo, bo):
  user = user.astype(jnp.int32)
  item = item.astype(jnp.int32)
  ug, ig, mu, mi = _sc_gather()(user, item, Ug.T, Ig.T, Um.T, Im.T)
  w1a = W1[:, :EMB].T
  w1b = W1[:, EMB:].T
  w2t = W2.T
  w3t = W3.T
  woa = Wo[:, :EMB].T
  wob = Wo[:, EMB:].T
  out = _mlp_call(ug, ig, mu, mi, w1a, w1b, w2t, w3t, woa, wob,
                  b1.reshape(1, 64), b2.reshape(1, 32), b3.reshape(1, 16),
                  bo.reshape(1, 1))
  return jnp.squeeze(out)
